# baseline (speedup 1.0000x reference)
import functools

import jax
import jax.numpy as jnp
from jax import lax
from jax.experimental import pallas as pl
from jax.experimental.pallas import tpu as pltpu
from jax.experimental.pallas import tpu_sc as plsc

_EMBED_DIM = 128
_CH = 64
_NFREQ = 32
_TABLE = 512
_LANES = 16


def _table_body(freq_ref, out_ref):
    freq2 = freq_ref[0:1, :]
    pos = lax.broadcasted_iota(jnp.int32, (_TABLE, _CH), 0).astype(jnp.float32)
    arg = pos * freq2
    odd = (lax.broadcasted_iota(jnp.int32, (_TABLE, _CH), 1) % 2) == 1
    out_ref[...] = jnp.where(odd, jnp.cos(arg), jnp.sin(arg))


def _build_table(freq_blk):
    return pl.pallas_call(
        _table_body,
        out_shape=jax.ShapeDtypeStruct((_TABLE, _CH), jnp.float32),
    )(freq_blk)


@functools.cache
def _sc_gather_call(n_idx):
    info = plsc.get_sparse_core_info()
    nc = info.num_cores
    nw = nc * info.num_subcores
    per_w = n_idx // nw
    per_o = per_w // 2
    n_out = n_idx // 2
    mesh = plsc.VectorSubcoreMesh(core_axis_name="c", subcore_axis_name="s")

    @functools.partial(
        pl.kernel,
        mesh=mesh,
        out_type=jax.ShapeDtypeStruct((n_out, _EMBED_DIM), jnp.float32),
        scratch_types=[
            pltpu.VMEM((per_o,), jnp.int32),
            pltpu.VMEM((per_o,), jnp.int32),
            pltpu.VMEM((per_o, _CH), jnp.float32),
            pltpu.VMEM((per_o, _CH), jnp.float32),
            pltpu.SemaphoreType.DMA,
            pltpu.SemaphoreType.DMA,
            pltpu.SemaphoreType.DMA,
        ],
        compiler_params=pltpu.CompilerParams(use_tc_tiling_on_sc=False),
    )
    def gather(table_hbm, h_hbm, w_hbm, out_hbm, idxh_v, idxw_v,
               hbuf, wbuf, sem_h, sem_w, sem_o):
        wid = lax.axis_index("s") * nc + lax.axis_index("c")
        base = wid * per_o
        pltpu.sync_copy(h_hbm.at[pl.ds(base, per_o)], idxh_v)
        pltpu.sync_copy(w_hbm.at[pl.ds(base, per_o)], idxw_v)
        ch = pltpu.async_copy(table_hbm.at[idxh_v], hbuf, sem_h)
        cw = pltpu.async_copy(table_hbm.at[idxw_v], wbuf, sem_w)
        ch.wait()
        wh = pltpu.async_copy(
            hbuf, out_hbm.at[pl.ds(base, per_o), pl.ds(0, _CH)], sem_o)
        cw.wait()
        ww = pltpu.async_copy(
            wbuf, out_hbm.at[pl.ds(base, per_o), pl.ds(_CH, _CH)], sem_o)
        wh.wait()
        ww.wait()

    return gather


def kernel(coord_idx, inv_freq):
    freq_blk = jnp.broadcast_to(jnp.repeat(inv_freq, 2)[None, :], (8, _CH))
    table = _build_table(freq_blk)
    n_idx = coord_idx.size
    flat2 = coord_idx.reshape(n_idx // 2, 2)
    return _sc_gather_call(n_idx)(table, flat2[:, 0], flat2[:, 1])


# speedup vs baseline: 1.2074x; 1.2074x over previous
import functools

import jax
import jax.numpy as jnp
from jax import lax
from jax.experimental import pallas as pl
from jax.experimental.pallas import tpu as pltpu
from jax.experimental.pallas import tpu_sc as plsc

_EMBED_DIM = 128
_CH = 64
_NFREQ = 32
_TABLE = 512
_LANES = 16


def _table_body(freq_ref, out_ref):
    freq2 = freq_ref[0:1, :]
    pos = lax.broadcasted_iota(jnp.int32, (_TABLE, _CH), 0).astype(jnp.float32)
    arg = pos * freq2
    odd = (lax.broadcasted_iota(jnp.int32, (_TABLE, _CH), 1) % 2) == 1
    out_ref[...] = jnp.where(odd, jnp.cos(arg), jnp.sin(arg))


def _build_table(freq_blk):
    return pl.pallas_call(
        _table_body,
        out_shape=jax.ShapeDtypeStruct((_TABLE, _CH), jnp.float32),
    )(freq_blk)


@functools.cache
def _sc_gather_call(n_idx):
    info = plsc.get_sparse_core_info()
    nc = info.num_cores
    nw = nc * info.num_subcores
    per_w = n_idx // nw
    per_o = per_w // 2
    n_out = n_idx // 2
    mesh = plsc.VectorSubcoreMesh(core_axis_name="c", subcore_axis_name="s")

    @functools.partial(
        pl.kernel,
        mesh=mesh,
        out_type=jax.ShapeDtypeStruct((n_out, _EMBED_DIM), jnp.float32),
        scratch_types=[
            pltpu.VMEM((per_o,), jnp.int32),
            pltpu.VMEM((per_o,), jnp.int32),
            pltpu.VMEM((per_o, _CH), jnp.float32),
            pltpu.VMEM((per_o, _CH), jnp.float32),
            pltpu.VMEM_SHARED((_TABLE, _CH), jnp.float32),
            pltpu.SemaphoreType.DMA,
            pltpu.SemaphoreType.DMA,
            pltpu.SemaphoreType.DMA,
        ],
        compiler_params=pltpu.CompilerParams(use_tc_tiling_on_sc=False),
    )
    def gather(table_hbm, h_hbm, w_hbm, out_hbm, idxh_v, idxw_v,
               hbuf, wbuf, table_sh, sem_h, sem_w, sem_o):
        wid = lax.axis_index("s") * nc + lax.axis_index("c")
        base = wid * per_o
        @pl.when(lax.axis_index("s") == 0)
        def _():
            pltpu.sync_copy(table_hbm, table_sh)
        pltpu.sync_copy(h_hbm.at[pl.ds(base, per_o)], idxh_v)
        pltpu.sync_copy(w_hbm.at[pl.ds(base, per_o)], idxw_v)
        plsc.subcore_barrier()
        ch = pltpu.async_copy(table_sh.at[idxh_v], hbuf, sem_h)
        cw = pltpu.async_copy(table_sh.at[idxw_v], wbuf, sem_w)
        ch.wait()
        wh = pltpu.async_copy(
            hbuf, out_hbm.at[pl.ds(base, per_o), pl.ds(0, _CH)], sem_o)
        cw.wait()
        ww = pltpu.async_copy(
            wbuf, out_hbm.at[pl.ds(base, per_o), pl.ds(_CH, _CH)], sem_o)
        wh.wait()
        ww.wait()

    return gather


def kernel(coord_idx, inv_freq):
    freq_blk = jnp.broadcast_to(jnp.repeat(inv_freq, 2)[None, :], (8, _CH))
    table = _build_table(freq_blk)
    n_idx = coord_idx.size
    flat2 = coord_idx.reshape(n_idx // 2, 2)
    return _sc_gather_call(n_idx)(table, flat2[:, 0], flat2[:, 1])


# speedup vs baseline: 1.2316x; 1.0200x over previous
import functools

import jax
import jax.numpy as jnp
from jax import lax
from jax.experimental import pallas as pl
from jax.experimental.pallas import tpu as pltpu
from jax.experimental.pallas import tpu_sc as plsc

_EMBED_DIM = 128
_CH = 64
_NFREQ = 32
_TABLE = 512
_LANES = 16


def _table_body(freq_ref, out_ref):
    freq2 = freq_ref[0:1, :]
    pos = lax.broadcasted_iota(jnp.int32, (_TABLE, _CH), 0).astype(jnp.float32)
    arg = pos * freq2
    odd = (lax.broadcasted_iota(jnp.int32, (_TABLE, _CH), 1) % 2) == 1
    out_ref[...] = jnp.where(odd, jnp.cos(arg), jnp.sin(arg))


def _build_table(freq_blk):
    return pl.pallas_call(
        _table_body,
        out_shape=jax.ShapeDtypeStruct((_TABLE, _CH), jnp.float32),
    )(freq_blk)


@functools.cache
def _sc_gather_call(n_idx):
    info = plsc.get_sparse_core_info()
    nc = info.num_cores
    nw = nc * info.num_subcores
    per_w = n_idx // nw
    per_o = per_w // 2
    n_out = n_idx // 2
    mesh = plsc.VectorSubcoreMesh(core_axis_name="c", subcore_axis_name="s")

    @functools.partial(
        pl.kernel,
        mesh=mesh,
        out_type=jax.ShapeDtypeStruct((n_out, _EMBED_DIM), jnp.float32),
        scratch_types=[
            pltpu.VMEM((per_w,), jnp.int32),
            pltpu.VMEM((per_w, _CH), jnp.float32),
            pltpu.VMEM_SHARED((_TABLE, _CH), jnp.float32),
            pltpu.SemaphoreType.DMA,
            pltpu.SemaphoreType.DMA,
        ],
        compiler_params=pltpu.CompilerParams(use_tc_tiling_on_sc=False),
    )
    def gather(table_hbm, idx_hbm, out_hbm, idx_v, buf, table_sh,
               sem_g, sem_o):
        wid = lax.axis_index("s") * nc + lax.axis_index("c")
        base = wid * per_o
        @pl.when(lax.axis_index("s") == 0)
        def _():
            pltpu.sync_copy(table_hbm, table_sh)
        pltpu.sync_copy(idx_hbm.at[pl.ds(wid * per_w, per_w)], idx_v)
        plsc.subcore_barrier()
        pltpu.async_copy(table_sh.at[idx_v], buf, sem_g).wait()
        wh = pltpu.async_copy(
            buf.at[pl.ds(0, per_o), :],
            out_hbm.at[pl.ds(base, per_o), pl.ds(0, _CH)], sem_o)
        ww = pltpu.async_copy(
            buf.at[pl.ds(per_o, per_o), :],
            out_hbm.at[pl.ds(base, per_o), pl.ds(_CH, _CH)], sem_o)
        wh.wait()
        ww.wait()

    return gather


def kernel(coord_idx, inv_freq):
    freq_blk = jnp.broadcast_to(jnp.repeat(inv_freq, 2)[None, :], (8, _CH))
    table = _build_table(freq_blk)
    n_idx = coord_idx.size
    n_out = n_idx // 2
    nw = 32
    per_o = n_out // nw
    hw = jnp.transpose(coord_idx.reshape(nw, per_o, 2), (0, 2, 1))
    return _sc_gather_call(n_idx)(table, hw.reshape(n_idx))


# speedup vs baseline: 1.2330x; 1.0011x over previous
import functools

import jax
import jax.numpy as jnp
from jax import lax
from jax.experimental import pallas as pl
from jax.experimental.pallas import tpu as pltpu
from jax.experimental.pallas import tpu_sc as plsc

_EMBED_DIM = 128
_CH = 64
_NFREQ = 32
_TABLE = 512
_LANES = 16


def _table_body(freq_ref, out_ref):
    freq2 = freq_ref[0:1, :]
    pos = lax.broadcasted_iota(jnp.int32, (_TABLE, _CH), 0).astype(jnp.float32)
    arg = pos * freq2
    odd = (lax.broadcasted_iota(jnp.int32, (_TABLE, _CH), 1) % 2) == 1
    out_ref[...] = jnp.where(odd, jnp.cos(arg), jnp.sin(arg))


def _build_table(freq_blk):
    return pl.pallas_call(
        _table_body,
        out_shape=jax.ShapeDtypeStruct((_TABLE, _CH), jnp.float32),
    )(freq_blk)


@functools.cache
def _sc_gather_call(n_idx):
    info = plsc.get_sparse_core_info()
    nc = info.num_cores
    nw = nc * info.num_subcores
    per_w = n_idx // nw
    per_o = per_w // 2
    n_out = n_idx // 2
    mesh = plsc.VectorSubcoreMesh(core_axis_name="c", subcore_axis_name="s")

    @functools.partial(
        pl.kernel,
        mesh=mesh,
        out_type=jax.ShapeDtypeStruct((n_out, _EMBED_DIM), jnp.float32),
        scratch_types=[
            pltpu.VMEM((per_w,), jnp.int32),
            pltpu.VMEM((per_w, _CH), jnp.float32),
            pltpu.VMEM_SHARED((_TABLE, _CH), jnp.float32),
            pltpu.SemaphoreType.DMA,
            pltpu.SemaphoreType.DMA,
        ],
        compiler_params=pltpu.CompilerParams(use_tc_tiling_on_sc=False),
    )
    def gather(table_hbm, idx_hbm, out_hbm, idx_v, buf, table_sh,
               sem_g, sem_o):
        wid = lax.axis_index("s") * nc + lax.axis_index("c")
        base = wid * per_o
        sid = lax.axis_index("s")
        half = _TABLE // 2

        @pl.when(sid < 2)
        def _():
            pltpu.sync_copy(table_hbm.at[pl.ds(sid * half, half)],
                            table_sh.at[pl.ds(sid * half, half)])
        pltpu.sync_copy(idx_hbm.at[pl.ds(wid * per_w, per_w)], idx_v)
        plsc.subcore_barrier()
        pltpu.async_copy(table_sh.at[idx_v], buf, sem_g).wait()
        wh = pltpu.async_copy(
            buf.at[pl.ds(0, per_o), :],
            out_hbm.at[pl.ds(base, per_o), pl.ds(0, _CH)], sem_o)
        ww = pltpu.async_copy(
            buf.at[pl.ds(per_o, per_o), :],
            out_hbm.at[pl.ds(base, per_o), pl.ds(_CH, _CH)], sem_o)
        wh.wait()
        ww.wait()

    return gather


def kernel(coord_idx, inv_freq):
    freq_blk = jnp.broadcast_to(jnp.repeat(inv_freq, 2)[None, :], (8, _CH))
    table = _build_table(freq_blk)
    n_idx = coord_idx.size
    n_out = n_idx // 2
    nw = 32
    per_o = n_out // nw
    hw = jnp.transpose(coord_idx.reshape(nw, per_o, 2), (0, 2, 1))
    return _sc_gather_call(n_idx)(table, hw.reshape(n_idx))
